# Initial kernel scaffold; baseline (speedup 1.0000x reference)
import jax
import jax.numpy as jnp
from jax.experimental import pallas as pl


def kernel(inp):
    raise NotImplementedError("write your pallas kernel here")


# speedup vs baseline: 53.8338x; 53.8338x over previous
import functools

import jax
import jax.numpy as jnp
from jax import lax
from jax.experimental import pallas as pl
from jax.experimental.pallas import tpu as pltpu
from jax.experimental.pallas import tpu_sc as plsc

ROWS = 128
COLS = 32768
TOTAL = ROWS * COLS
K = 10
BIG = 0x7FFFFFFF

_info = plsc.get_sparse_core_info()
NC = _info.num_cores
NS = _info.num_subcores
L = _info.num_lanes
NW = NC * NS
PER_W = TOTAL // NW
HALVES = 2
HALF = PER_W // HALVES
VECS = HALF // L
UNROLL = 8
RESCAN_VECS = VECS // L
RESCAN_UNROLL = 4
NSLOT = NW * HALVES


def _lane_argmax(m, ix, iota):
    for s in (8, 4, 2, 1):
        perm = iota ^ s
        om = m.at[perm].get(mode="promise_in_bounds")
        oi = ix.at[perm].get(mode="promise_in_bounds")
        take = (om > m) | ((om == m) & (oi < ix))
        m = jnp.where(take, om, m)
        ix = jnp.where(take, oi, ix)
    return m, ix


def _sc_topk_body(inp_ref, cv_ref, ci_ref, sm_ref, buf, stage_v, stage_i,
                  stage_s):
    iota = lax.iota(jnp.int32, L)
    wid = lax.axis_index("s") * NC + lax.axis_index("c")
    sum_acc = jnp.zeros((L,), jnp.float32)

    for h in range(HALVES):
        base = wid * PER_W + h * HALF
        pltpu.sync_copy(inp_ref.at[pl.ds(base, HALF)], buf)

        def scan_step(i, carry):
            m, idx, vsum, ivec = carry
            for u in range(UNROLL):
                v = buf[pl.ds(i * (L * UNROLL) + u * L, L)]
                upd = v > m
                m = jnp.where(upd, v, m)
                idx = jnp.where(upd, ivec, idx)
                vsum = vsum + v
                ivec = ivec + L
            return m, idx, vsum, ivec

        m0 = jnp.full((L,), -1.0, jnp.float32)
        i0 = jnp.full((L,), BIG, jnp.int32)
        s0 = jnp.zeros((L,), jnp.float32)
        v0 = base + iota
        m, idx, vsum, _ = lax.fori_loop(0, VECS // UNROLL, scan_step,
                                        (m0, i0, s0, v0))
        sum_acc = sum_acc + vsum

        cand_v = jnp.full((L,), -1.0, jnp.float32)
        cand_i = jnp.zeros((L,), jnp.int32)
        for r in range(K):
            gmax, cidx = _lane_argmax(m, idx, iota)
            cand_v = jnp.where(iota == r, gmax, cand_v)
            cand_i = jnp.where(iota == r, cidx, cand_i)
            if r < K - 1:
                loc = cidx - base
                plsc.store_scatter(buf, [loc],
                                   jnp.full((L,), -1.0, jnp.float32),
                                   mask=iota == 0)
                lane = loc & (L - 1)

                def rescan_step(k, carry):
                    ml, il = carry
                    for u in range(RESCAN_UNROLL):
                        kk = k * RESCAN_UNROLL + u
                        pos = lane + kk * (L * L) + iota * L
                        v = plsc.load_gather(buf, [pos])
                        upd = v > ml
                        ml = jnp.where(upd, v, ml)
                        il = jnp.where(upd, pos, il)
                    return ml, il

                ml0 = jnp.full((L,), -1.0, jnp.float32)
                il0 = jnp.full((L,), BIG, jnp.int32)
                ml, il = lax.fori_loop(0, RESCAN_VECS // RESCAN_UNROLL,
                                       rescan_step, (ml0, il0))
                lmax, lloc = _lane_argmax(ml, il, iota)
                m = jnp.where(iota == lane, lmax, m)
                idx = jnp.where(iota == lane, base + lloc, idx)

        slot = wid * HALVES + h
        stage_v[...] = cand_v
        stage_i[...] = cand_i
        pltpu.sync_copy(stage_v, cv_ref.at[slot])
        pltpu.sync_copy(stage_i, ci_ref.at[slot])

    stage_s[...] = sum_acc
    pltpu.sync_copy(stage_s, sm_ref.at[wid])


_sc_topk = functools.partial(
    pl.kernel,
    out_type=(
        jax.ShapeDtypeStruct((NSLOT, L), jnp.float32),
        jax.ShapeDtypeStruct((NSLOT, L), jnp.int32),
        jax.ShapeDtypeStruct((NW, L), jnp.float32),
    ),
    mesh=plsc.VectorSubcoreMesh(core_axis_name="c", subcore_axis_name="s"),
    compiler_params=pltpu.CompilerParams(needs_layout_passes=False),
    scratch_types=[
        pltpu.VMEM((HALF,), jnp.float32),
        pltpu.VMEM((L,), jnp.float32),
        pltpu.VMEM((L,), jnp.int32),
        pltpu.VMEM((L,), jnp.float32),
    ],
)(_sc_topk_body)


def _merge_body(v_ref, i_ref, s_ref, o_ref):
    v = v_ref[...]
    ix = i_ref[...]
    total = jnp.sum(s_ref[...])
    kt = jnp.minimum(jnp.float32(K), total).astype(jnp.int32)
    for r in range(K):
        gmax = jnp.max(v)
        gi = jnp.min(jnp.where(v == gmax, ix, BIG))
        valid = r < kt
        row = lax.shift_right_logical(gi, 15)
        col = gi & (COLS - 1)
        o_ref[r, 0] = jnp.where(valid, row, 0)
        o_ref[r, 1] = jnp.where(valid, col, 0)
        v = jnp.where((ix == gi) & (v == gmax), jnp.float32(-1.0), v)


def kernel(inp):
    flat = inp.astype(jnp.float32).reshape(-1)
    cv, ci, sm = _sc_topk(flat)
    cv = cv.reshape(8, 128)
    ci = ci.reshape(8, 128)
    sm = jnp.concatenate([sm.reshape(4, 128),
                          jnp.zeros((4, 128), jnp.float32)], axis=0)
    out = pl.pallas_call(
        _merge_body,
        out_shape=jax.ShapeDtypeStruct((K, 2), jnp.int32),
        out_specs=pl.BlockSpec(memory_space=pltpu.SMEM),
    )(cv, ci, sm)
    return out.astype(jnp.int64)
